# baseline (speedup 1.0000x reference)
import functools

import jax
import jax.numpy as jnp
import numpy as np
from jax import lax
from jax.experimental import pallas as pl
from jax.experimental.pallas import tpu as pltpu
from jax.experimental.pallas import tpu_sc as plsc

Q, D = 3136, 1024
K = 8192
B, H, W = 4, 28, 28
HW = H * W
QB = 448
KB = 2048
NQB = Q // QB
NKB = K // KB
NG = KB // 128
TARGET = 224


def _build_A() -> np.ndarray:
    scale = TARGET / H
    s = (np.arange(TARGET) + 0.5) / scale - 0.5
    R = np.maximum(0.0, 1.0 - np.abs(s[:, None] - np.arange(H)[None, :]))
    R = R / R.sum(axis=1, keepdims=True)
    t = np.arange(-16, 17, dtype=np.float64)
    g = np.exp(-0.5 * (t / 4.0) ** 2)
    g = g / g.sum()
    G = np.zeros((TARGET, TARGET))
    for u in range(-16, 17):
        d = np.diagonal(G, offset=u)
        d.setflags(write=True)
        d[:] = g[u + 16]
    return (G @ R).astype(np.float32)


_A_NP = _build_A()


def _k2_kernel(k_ref, k2_ref):
    k = k_ref[...]
    k2_ref[0, :] = jnp.sum(k * k, axis=1)


def _key_norms(keys):
    return pl.pallas_call(
        _k2_kernel,
        grid=(NKB,),
        in_specs=[pl.BlockSpec((KB, D), lambda j: (j, 0))],
        out_specs=pl.BlockSpec((1, KB), lambda j: (0, j)),
        out_shape=jax.ShapeDtypeStruct((1, K), jnp.float32),
    )(keys)


def _dist_kernel(q_ref, k_ref, k2_ref, mval_ref, marg_ref, mval, marg):
    kb = pl.program_id(1)

    s = lax.dot_general(q_ref[...] * -2.0, k_ref[...],
                        dimension_numbers=(((1,), (1,)), ((), ())),
                        preferred_element_type=jnp.float32)
    val = s + k2_ref[0, :][None, :]

    vmin = val[:, 0:128]
    varg = jnp.zeros((QB, 128), jnp.int32)
    for g in range(1, NG):
        vg = val[:, 128 * g:128 * (g + 1)]
        upd = vg < vmin
        vmin = jnp.where(upd, vg, vmin)
        varg = jnp.where(upd, jnp.full((QB, 128), g, jnp.int32), varg)
    varg = varg + kb * NG

    @pl.when(kb == 0)
    def _():
        mval[...] = vmin
        marg[...] = varg

    @pl.when(kb > 0)
    def _():
        mv = mval[...]
        upd = vmin < mv
        mval[...] = jnp.where(upd, vmin, mv)
        marg[...] = jnp.where(upd, varg, marg[...])

    @pl.when(kb == NKB - 1)
    def _():
        mval_ref[0] = mval[...]
        marg_ref[0] = marg[...]


def _distances(qp, keys, k2):
    return pl.pallas_call(
        _dist_kernel,
        grid=(NQB, NKB),
        in_specs=[
            pl.BlockSpec((QB, D), lambda i, j: (i, 0)),
            pl.BlockSpec((KB, D), lambda i, j: (j, 0)),
            pl.BlockSpec((1, KB), lambda i, j: (0, j)),
        ],
        out_specs=[
            pl.BlockSpec((1, QB, 128), lambda i, j: (i, 0, 0)),
            pl.BlockSpec((1, QB, 128), lambda i, j: (i, 0, 0)),
        ],
        out_shape=[
            jax.ShapeDtypeStruct((NQB, QB, 128), jnp.float32),
            jax.ShapeDtypeStruct((NQB, QB, 128), jnp.int32),
        ],
        scratch_shapes=[
            pltpu.VMEM((QB, 128), jnp.float32),
            pltpu.VMEM((QB, 128), jnp.int32),
        ],
    )(qp, keys, k2)


def _epilogue_kernel(q_ref, mval_ref, marg_ref, dist_ref, am_ref, fidx_ref):
    qi = pl.program_id(0)
    mv = mval_ref[0]
    mg = marg_ref[0]
    m = jnp.min(mv, axis=1)
    lane = lax.broadcasted_iota(jnp.int32, (QB, 128), 1)
    flat = mg * 128 + lane
    am = jnp.min(jnp.where(mv == m[:, None], flat, jnp.int32(2**30)), axis=1)
    q = q_ref[...]
    q2 = jnp.sum(q * q, axis=1)
    dist_ref[0, 0, :] = q2 + m
    am_ref[0, 0, :] = am
    qid = qi * QB + lax.broadcasted_iota(jnp.int32, (QB,), 0)
    fidx_ref[0, 0, :] = (qid % HW) * (K // 16) + am // 16


def _epilogue(qp, mval, marg):
    return pl.pallas_call(
        _epilogue_kernel,
        grid=(NQB,),
        in_specs=[
            pl.BlockSpec((QB, D), lambda i: (i, 0)),
            pl.BlockSpec((1, QB, 128), lambda i: (i, 0, 0)),
            pl.BlockSpec((1, QB, 128), lambda i: (i, 0, 0)),
        ],
        out_specs=[
            pl.BlockSpec((1, 1, QB), lambda i: (i, 0, 0)),
            pl.BlockSpec((1, 1, QB), lambda i: (i, 0, 0)),
            pl.BlockSpec((1, 1, QB), lambda i: (i, 0, 0)),
        ],
        out_shape=[
            jax.ShapeDtypeStruct((NQB, 1, QB), jnp.float32),
            jax.ShapeDtypeStruct((NQB, 1, QB), jnp.int32),
            jax.ShapeDtypeStruct((NQB, 1, QB), jnp.int32),
        ],
    )(qp, mval, marg)


_NC, _NS = 2, 16
_NW = _NC * _NS
GPAD = 3584
_PER_W = GPAD // _NW


def _sc_gather_kernel(table_hbm, fidx_hbm, out_hbm, idx_v, rows_v, sem):
    wid = lax.axis_index("s") * _NC + lax.axis_index("c")
    base = wid * _PER_W
    pltpu.sync_copy(fidx_hbm.at[pl.ds(base, _PER_W)], idx_v)
    pltpu.async_copy(table_hbm.at[idx_v], rows_v, sem).wait()
    pltpu.sync_copy(rows_v, out_hbm.at[pl.ds(base, _PER_W)])


def _sc_gather(table16, fidx_pad):
    mesh = plsc.VectorSubcoreMesh(core_axis_name="c", subcore_axis_name="s")
    f = functools.partial(
        pl.kernel,
        mesh=mesh,
        out_type=jax.ShapeDtypeStruct((GPAD, 16), jnp.float32),
        scratch_types=[
            pltpu.VMEM((_PER_W,), jnp.int32),
            pltpu.VMEM((_PER_W, 16), jnp.float32),
            pltpu.SemaphoreType.DMA,
        ],
        compiler_params=pltpu.CompilerParams(use_tc_tiling_on_sc=False),
    )(_sc_gather_kernel)
    return f(table16, fidx_pad)


def _post_kernel(dist_ref, rows_ref, am_ref, a_ref, score_ref, ds_ref, ps_ref):
    a = a_ref[...]
    x = dist_ref[0]
    rows = rows_ref[0]
    l16 = am_ref[0, 0] % 16
    lane = lax.broadcasted_iota(jnp.int32, (HW, 16), 1)
    p784 = jnp.sum(jnp.where(lane == l16[:, None], rows, 0.0), axis=1)
    p = p784.reshape(H, W)
    score_ref[...] = jnp.full((1, 1, 1), jnp.max(x), jnp.float32)
    for src, dst in ((x, ds_ref), (p, ps_ref)):
        t = lax.dot_general(a, src, dimension_numbers=(((1,), (0,)), ((), ())),
                            preferred_element_type=jnp.float32,
                            precision=lax.Precision.HIGHEST)
        o = lax.dot_general(t, a, dimension_numbers=(((1,), (1,)), ((), ())),
                            preferred_element_type=jnp.float32,
                            precision=lax.Precision.HIGHEST)
        mn = jnp.min(o)
        mx = jnp.max(o)
        dst[0] = (o - mn) / (mx - mn)


def _postprocess(dist3, rows3, am3, a_mat):
    return pl.pallas_call(
        _post_kernel,
        grid=(B,),
        in_specs=[
            pl.BlockSpec((1, H, W), lambda i: (i, 0, 0)),
            pl.BlockSpec((1, HW, 16), lambda i: (i, 0, 0)),
            pl.BlockSpec((1, 1, HW), lambda i: (i, 0, 0)),
            pl.BlockSpec((TARGET, H), lambda i: (0, 0)),
        ],
        out_specs=[
            pl.BlockSpec((1, 1, 1), lambda i: (i, 0, 0)),
            pl.BlockSpec((1, TARGET, TARGET), lambda i: (i, 0, 0)),
            pl.BlockSpec((1, TARGET, TARGET), lambda i: (i, 0, 0)),
        ],
        out_shape=[
            jax.ShapeDtypeStruct((B, 1, 1), jnp.float32),
            jax.ShapeDtypeStruct((B, TARGET, TARGET), jnp.float32),
            jax.ShapeDtypeStruct((B, TARGET, TARGET), jnp.float32),
        ],
    )(dist3, rows3, am3, a_mat)


def kernel(queries, keys, similar_map):
    k2 = _key_norms(keys)
    mval, marg = _distances(queries, keys, k2)
    dist5, am5, fidx5 = _epilogue(queries, mval, marg)
    dist = dist5.reshape(Q)
    fidx = fidx5.reshape(Q)
    fidx_pad = jnp.pad(fidx, (0, GPAD - Q))
    table16 = similar_map.transpose(1, 2, 0).reshape(HW * K // 16, 16)
    rows = _sc_gather(table16, fidx_pad)[:Q]
    dist3 = dist.reshape(B, H, W)
    rows3 = rows.reshape(B, HW, 16)
    am3 = am5.reshape(Q).reshape(B, 1, HW)
    a_mat = jnp.asarray(_A_NP)
    score3, ds, ps = _postprocess(dist3, rows3, am3, a_mat)
    return (score3.reshape(B), ds, ps)


# speedup vs baseline: 1.1152x; 1.1152x over previous
import functools

import jax
import jax.numpy as jnp
import numpy as np
from jax import lax
from jax.experimental import pallas as pl
from jax.experimental.pallas import tpu as pltpu
from jax.experimental.pallas import tpu_sc as plsc

Q, D = 3136, 1024
K = 8192
B, H, W = 4, 28, 28
HW = H * W
QB = 448
KB = 2048
NQB = Q // QB
NKB = K // KB
NG = KB // 128
TARGET = 224


def _build_A() -> np.ndarray:
    scale = TARGET / H
    s = (np.arange(TARGET) + 0.5) / scale - 0.5
    R = np.maximum(0.0, 1.0 - np.abs(s[:, None] - np.arange(H)[None, :]))
    R = R / R.sum(axis=1, keepdims=True)
    t = np.arange(-16, 17, dtype=np.float64)
    g = np.exp(-0.5 * (t / 4.0) ** 2)
    g = g / g.sum()
    G = np.zeros((TARGET, TARGET))
    for u in range(-16, 17):
        d = np.diagonal(G, offset=u)
        d.setflags(write=True)
        d[:] = g[u + 16]
    return (G @ R).astype(np.float32)


_A_NP = _build_A()


def _dist_kernel(q_ref, k_ref, dist_ref, am_ref, fidx_ref, k2s, mval, marg):
    qi = pl.program_id(0)
    kb = pl.program_id(1)

    @pl.when(qi == 0)
    def _():
        k = k_ref[...]
        k2s[0, pl.ds(kb * KB, KB)] = jnp.sum(k * k, axis=1)

    s = lax.dot_general(q_ref[...] * -2.0, k_ref[...],
                        dimension_numbers=(((1,), (1,)), ((), ())),
                        preferred_element_type=jnp.float32)
    val = s + k2s[0, pl.ds(kb * KB, KB)][None, :]

    vmin = val[:, 0:128]
    varg = jnp.zeros((QB, 128), jnp.int32)
    for g in range(1, NG):
        vg = val[:, 128 * g:128 * (g + 1)]
        upd = vg < vmin
        vmin = jnp.where(upd, vg, vmin)
        varg = jnp.where(upd, jnp.full((QB, 128), g, jnp.int32), varg)
    varg = varg + kb * NG

    @pl.when(kb == 0)
    def _():
        mval[...] = vmin
        marg[...] = varg

    @pl.when(kb > 0)
    def _():
        mv = mval[...]
        upd = vmin < mv
        mval[...] = jnp.where(upd, vmin, mv)
        marg[...] = jnp.where(upd, varg, marg[...])

    @pl.when(kb == NKB - 1)
    def _():
        mv = mval[...]
        mg = marg[...]
        m = jnp.min(mv, axis=1)
        lane = lax.broadcasted_iota(jnp.int32, (QB, 128), 1)
        flat = mg * 128 + lane
        am = jnp.min(jnp.where(mv == m[:, None], flat, jnp.int32(2**30)),
                     axis=1)
        q = q_ref[...]
        q2 = jnp.sum(q * q, axis=1)
        dist_ref[0, 0, :] = q2 + m
        am_ref[0, 0, :] = am
        qid = qi * QB + lax.broadcasted_iota(jnp.int32, (QB,), 0)
        fidx_ref[0, 0, :] = (qid % HW) * (K // 16) + am // 16


def _distances(qp, keys):
    return pl.pallas_call(
        _dist_kernel,
        grid=(NQB, NKB),
        in_specs=[
            pl.BlockSpec((QB, D), lambda i, j: (i, 0)),
            pl.BlockSpec((KB, D), lambda i, j: (j, 0)),
        ],
        out_specs=[
            pl.BlockSpec((1, 1, QB), lambda i, j: (i, 0, 0)),
            pl.BlockSpec((1, 1, QB), lambda i, j: (i, 0, 0)),
            pl.BlockSpec((1, 1, QB), lambda i, j: (i, 0, 0)),
        ],
        out_shape=[
            jax.ShapeDtypeStruct((NQB, 1, QB), jnp.float32),
            jax.ShapeDtypeStruct((NQB, 1, QB), jnp.int32),
            jax.ShapeDtypeStruct((NQB, 1, QB), jnp.int32),
        ],
        scratch_shapes=[
            pltpu.VMEM((1, K), jnp.float32),
            pltpu.VMEM((QB, 128), jnp.float32),
            pltpu.VMEM((QB, 128), jnp.int32),
        ],
    )(qp, keys)


_NC, _NS = 2, 16
_NW = _NC * _NS
GPAD = 3584
_PER_W = GPAD // _NW


def _sc_gather_kernel(table_hbm, fidx_hbm, out_hbm, idx_v, rows_v, sem):
    wid = lax.axis_index("s") * _NC + lax.axis_index("c")
    base = wid * _PER_W
    pltpu.sync_copy(fidx_hbm.at[pl.ds(base, _PER_W)], idx_v)
    pltpu.async_copy(table_hbm.at[idx_v], rows_v, sem).wait()
    pltpu.sync_copy(rows_v, out_hbm.at[pl.ds(base, _PER_W)])


def _sc_gather(table16, fidx_pad):
    mesh = plsc.VectorSubcoreMesh(core_axis_name="c", subcore_axis_name="s")
    f = functools.partial(
        pl.kernel,
        mesh=mesh,
        out_type=jax.ShapeDtypeStruct((GPAD, 16), jnp.float32),
        scratch_types=[
            pltpu.VMEM((_PER_W,), jnp.int32),
            pltpu.VMEM((_PER_W, 16), jnp.float32),
            pltpu.SemaphoreType.DMA,
        ],
        compiler_params=pltpu.CompilerParams(use_tc_tiling_on_sc=False),
    )(_sc_gather_kernel)
    return f(table16, fidx_pad)


def _post_kernel(dist_ref, rows_ref, am_ref, a_ref, score_ref, ds_ref, ps_ref):
    a = a_ref[...]
    x = dist_ref[0]
    rows = rows_ref[0]
    l16 = am_ref[0] % 16
    lane = lax.broadcasted_iota(jnp.int32, (H, W, 16), 2)
    p = jnp.sum(jnp.where(lane == l16[:, :, None], rows, 0.0), axis=2)
    score_ref[...] = jnp.full((1, 1, 1), jnp.max(x), jnp.float32)
    for src, dst in ((x, ds_ref), (p, ps_ref)):
        t = lax.dot_general(a, src, dimension_numbers=(((1,), (0,)), ((), ())),
                            preferred_element_type=jnp.float32,
                            precision=lax.Precision.HIGHEST)
        o = lax.dot_general(t, a, dimension_numbers=(((1,), (1,)), ((), ())),
                            preferred_element_type=jnp.float32,
                            precision=lax.Precision.HIGHEST)
        mn = jnp.min(o)
        mx = jnp.max(o)
        dst[0] = (o - mn) / (mx - mn)


def _postprocess(dist3, rows3, am3, a_mat):
    return pl.pallas_call(
        _post_kernel,
        grid=(B,),
        in_specs=[
            pl.BlockSpec((1, H, W), lambda i: (i, 0, 0)),
            pl.BlockSpec((1, H, W, 16), lambda i: (i, 0, 0, 0)),
            pl.BlockSpec((1, H, W), lambda i: (i, 0, 0)),
            pl.BlockSpec((TARGET, H), lambda i: (0, 0)),
        ],
        out_specs=[
            pl.BlockSpec((1, 1, 1), lambda i: (i, 0, 0)),
            pl.BlockSpec((1, TARGET, TARGET), lambda i: (i, 0, 0)),
            pl.BlockSpec((1, TARGET, TARGET), lambda i: (i, 0, 0)),
        ],
        out_shape=[
            jax.ShapeDtypeStruct((B, 1, 1), jnp.float32),
            jax.ShapeDtypeStruct((B, TARGET, TARGET), jnp.float32),
            jax.ShapeDtypeStruct((B, TARGET, TARGET), jnp.float32),
        ],
    )(dist3, rows3, am3, a_mat)


def kernel(queries, keys, similar_map):
    dist5, am5, fidx5 = _distances(queries, keys)
    dist = dist5.reshape(Q)
    fidx = fidx5.reshape(Q)
    fidx_pad = jnp.pad(fidx, (0, GPAD - Q))
    table16 = similar_map.transpose(1, 2, 0).reshape(HW * K // 16, 16)
    rows = _sc_gather(table16, fidx_pad)[:Q]
    dist3 = dist.reshape(B, H, W)
    rows3 = rows.reshape(B, H, W, 16)
    am3 = am5.reshape(Q).reshape(B, H, W)
    a_mat = jnp.asarray(_A_NP)
    score3, ds, ps = _postprocess(dist3, rows3, am3, a_mat)
    return (score3.reshape(B), ds, ps)


# speedup vs baseline: 1.2510x; 1.1217x over previous
import functools

import jax
import jax.numpy as jnp
import numpy as np
from jax import lax
from jax.experimental import pallas as pl
from jax.experimental.pallas import tpu as pltpu
from jax.experimental.pallas import tpu_sc as plsc

Q, D = 3136, 1024
K = 8192
B, H, W = 4, 28, 28
HW = H * W
QB = 1568
KB = 2048
NQB = Q // QB
NKB = K // KB
NG = KB // 128
TARGET = 224


def _build_A() -> np.ndarray:
    scale = TARGET / H
    s = (np.arange(TARGET) + 0.5) / scale - 0.5
    R = np.maximum(0.0, 1.0 - np.abs(s[:, None] - np.arange(H)[None, :]))
    R = R / R.sum(axis=1, keepdims=True)
    t = np.arange(-16, 17, dtype=np.float64)
    g = np.exp(-0.5 * (t / 4.0) ** 2)
    g = g / g.sum()
    G = np.zeros((TARGET, TARGET))
    for u in range(-16, 17):
        d = np.diagonal(G, offset=u)
        d.setflags(write=True)
        d[:] = g[u + 16]
    return (G @ R).astype(np.float32)


_A_NP = _build_A()


def _dist_kernel(q_ref, k_ref, dist_ref, am_ref, fidx_ref, k2s, mval, marg):
    qi = pl.program_id(0)
    kb = pl.program_id(1)

    @pl.when(qi == 0)
    def _():
        k = k_ref[...]
        k2s[0, pl.ds(kb * KB, KB)] = jnp.sum(k * k, axis=1)

    s = lax.dot_general(q_ref[...] * -2.0, k_ref[...],
                        dimension_numbers=(((1,), (1,)), ((), ())),
                        preferred_element_type=jnp.float32)
    val = s + k2s[0, pl.ds(kb * KB, KB)][None, :]

    vmin = val[:, 0:128]
    varg = jnp.zeros((QB, 128), jnp.int32)
    for g in range(1, NG):
        vg = val[:, 128 * g:128 * (g + 1)]
        upd = vg < vmin
        vmin = jnp.where(upd, vg, vmin)
        varg = jnp.where(upd, jnp.full((QB, 128), g, jnp.int32), varg)
    varg = varg + kb * NG

    @pl.when(kb == 0)
    def _():
        mval[...] = vmin
        marg[...] = varg

    @pl.when(kb > 0)
    def _():
        mv = mval[...]
        upd = vmin < mv
        mval[...] = jnp.where(upd, vmin, mv)
        marg[...] = jnp.where(upd, varg, marg[...])

    @pl.when(kb == NKB - 1)
    def _():
        mv = mval[...]
        mg = marg[...]
        m = jnp.min(mv, axis=1)
        lane = lax.broadcasted_iota(jnp.int32, (QB, 128), 1)
        flat = mg * 128 + lane
        am = jnp.min(jnp.where(mv == m[:, None], flat, jnp.int32(2**30)),
                     axis=1)
        q = q_ref[...]
        q2 = jnp.sum(q * q, axis=1)
        dist_ref[0, 0, :] = q2 + m
        am_ref[0, 0, :] = am
        qid = qi * QB + lax.broadcasted_iota(jnp.int32, (QB,), 0)
        fidx_ref[0, 0, :] = (qid % HW) * (K // 16) + am // 16


def _distances(qp, keys):
    return pl.pallas_call(
        _dist_kernel,
        grid=(NQB, NKB),
        in_specs=[
            pl.BlockSpec((QB, D), lambda i, j: (i, 0)),
            pl.BlockSpec((KB, D), lambda i, j: (j, 0)),
        ],
        out_specs=[
            pl.BlockSpec((1, 1, QB), lambda i, j: (i, 0, 0)),
            pl.BlockSpec((1, 1, QB), lambda i, j: (i, 0, 0)),
            pl.BlockSpec((1, 1, QB), lambda i, j: (i, 0, 0)),
        ],
        out_shape=[
            jax.ShapeDtypeStruct((NQB, 1, QB), jnp.float32),
            jax.ShapeDtypeStruct((NQB, 1, QB), jnp.int32),
            jax.ShapeDtypeStruct((NQB, 1, QB), jnp.int32),
        ],
        scratch_shapes=[
            pltpu.VMEM((1, K), jnp.float32),
            pltpu.VMEM((QB, 128), jnp.float32),
            pltpu.VMEM((QB, 128), jnp.int32),
        ],
    )(qp, keys)


_NC, _NS = 2, 16
_NW = _NC * _NS
GPAD = 3584
_PER_W = GPAD // _NW


def _sc_gather_kernel(table_hbm, fidx_hbm, out_hbm, idx_v, rows_v, sem):
    wid = lax.axis_index("s") * _NC + lax.axis_index("c")
    base = wid * _PER_W
    pltpu.sync_copy(fidx_hbm.at[pl.ds(base, _PER_W)], idx_v)
    pltpu.async_copy(table_hbm.at[idx_v], rows_v, sem).wait()
    pltpu.sync_copy(rows_v, out_hbm.at[pl.ds(base, _PER_W)])


def _sc_gather(table16, fidx_pad):
    mesh = plsc.VectorSubcoreMesh(core_axis_name="c", subcore_axis_name="s")
    f = functools.partial(
        pl.kernel,
        mesh=mesh,
        out_type=jax.ShapeDtypeStruct((GPAD, 16), jnp.float32),
        scratch_types=[
            pltpu.VMEM((_PER_W,), jnp.int32),
            pltpu.VMEM((_PER_W, 16), jnp.float32),
            pltpu.SemaphoreType.DMA,
        ],
        compiler_params=pltpu.CompilerParams(use_tc_tiling_on_sc=False),
    )(_sc_gather_kernel)
    return f(table16, fidx_pad)


def _post_kernel(dist_ref, rows_ref, am_ref, a_ref, score_ref, ds_ref, ps_ref):
    a = a_ref[...]
    x = dist_ref[0]
    rows = rows_ref[0]
    l16 = am_ref[0] % 16
    lane = lax.broadcasted_iota(jnp.int32, (H, W, 16), 2)
    p = jnp.sum(jnp.where(lane == l16[:, :, None], rows, 0.0), axis=2)
    score_ref[...] = jnp.full((1, 1, 1), jnp.max(x), jnp.float32)
    for src, dst in ((x, ds_ref), (p, ps_ref)):
        t = lax.dot_general(a, src, dimension_numbers=(((1,), (0,)), ((), ())),
                            preferred_element_type=jnp.float32,
                            precision=lax.Precision.HIGHEST)
        o = lax.dot_general(t, a, dimension_numbers=(((1,), (1,)), ((), ())),
                            preferred_element_type=jnp.float32,
                            precision=lax.Precision.HIGHEST)
        mn = jnp.min(o)
        mx = jnp.max(o)
        dst[0] = (o - mn) / (mx - mn)


def _postprocess(dist3, rows3, am3, a_mat):
    return pl.pallas_call(
        _post_kernel,
        grid=(B,),
        in_specs=[
            pl.BlockSpec((1, H, W), lambda i: (i, 0, 0)),
            pl.BlockSpec((1, H, W, 16), lambda i: (i, 0, 0, 0)),
            pl.BlockSpec((1, H, W), lambda i: (i, 0, 0)),
            pl.BlockSpec((TARGET, H), lambda i: (0, 0)),
        ],
        out_specs=[
            pl.BlockSpec((1, 1, 1), lambda i: (i, 0, 0)),
            pl.BlockSpec((1, TARGET, TARGET), lambda i: (i, 0, 0)),
            pl.BlockSpec((1, TARGET, TARGET), lambda i: (i, 0, 0)),
        ],
        out_shape=[
            jax.ShapeDtypeStruct((B, 1, 1), jnp.float32),
            jax.ShapeDtypeStruct((B, TARGET, TARGET), jnp.float32),
            jax.ShapeDtypeStruct((B, TARGET, TARGET), jnp.float32),
        ],
    )(dist3, rows3, am3, a_mat)


def kernel(queries, keys, similar_map):
    dist5, am5, fidx5 = _distances(queries, keys)
    dist = dist5.reshape(Q)
    fidx = fidx5.reshape(Q)
    fidx_pad = jnp.pad(fidx, (0, GPAD - Q))
    table16 = similar_map.transpose(1, 2, 0).reshape(HW * K // 16, 16)
    rows = _sc_gather(table16, fidx_pad)[:Q]
    dist3 = dist.reshape(B, H, W)
    rows3 = rows.reshape(B, H, W, 16)
    am3 = am5.reshape(Q).reshape(B, H, W)
    a_mat = jnp.asarray(_A_NP)
    score3, ds, ps = _postprocess(dist3, rows3, am3, a_mat)
    return (score3.reshape(B), ds, ps)


# speedup vs baseline: 1.2958x; 1.0358x over previous
import functools

import jax
import jax.numpy as jnp
import numpy as np
from jax import lax
from jax.experimental import pallas as pl
from jax.experimental.pallas import tpu as pltpu
from jax.experimental.pallas import tpu_sc as plsc

Q, D = 3136, 1024
K = 8192
B, H, W = 4, 28, 28
HW = H * W
QB = 3136
KB = 2048
NQB = Q // QB
NKB = K // KB
NG = KB // 128
TARGET = 224


def _build_A() -> np.ndarray:
    scale = TARGET / H
    s = (np.arange(TARGET) + 0.5) / scale - 0.5
    R = np.maximum(0.0, 1.0 - np.abs(s[:, None] - np.arange(H)[None, :]))
    R = R / R.sum(axis=1, keepdims=True)
    t = np.arange(-16, 17, dtype=np.float64)
    g = np.exp(-0.5 * (t / 4.0) ** 2)
    g = g / g.sum()
    G = np.zeros((TARGET, TARGET))
    for u in range(-16, 17):
        d = np.diagonal(G, offset=u)
        d.setflags(write=True)
        d[:] = g[u + 16]
    return (G @ R).astype(np.float32)


_A_NP = _build_A()


def _dist_kernel(q_ref, k_ref, dist_ref, am_ref, fidx_ref, k2s, mval, marg):
    qi = pl.program_id(0)
    kb = pl.program_id(1)

    @pl.when(qi == 0)
    def _():
        k = k_ref[...]
        k2s[0, pl.ds(kb * KB, KB)] = jnp.sum(k * k, axis=1)

    s = lax.dot_general(q_ref[...] * -2.0, k_ref[...],
                        dimension_numbers=(((1,), (1,)), ((), ())),
                        preferred_element_type=jnp.float32)
    val = s + k2s[0, pl.ds(kb * KB, KB)][None, :]

    vmin = val[:, 0:128]
    varg = jnp.zeros((QB, 128), jnp.int32)
    for g in range(1, NG):
        vg = val[:, 128 * g:128 * (g + 1)]
        upd = vg < vmin
        vmin = jnp.where(upd, vg, vmin)
        varg = jnp.where(upd, jnp.full((QB, 128), g, jnp.int32), varg)
    varg = varg + kb * NG

    @pl.when(kb == 0)
    def _():
        mval[...] = vmin
        marg[...] = varg

    @pl.when(kb > 0)
    def _():
        mv = mval[...]
        upd = vmin < mv
        mval[...] = jnp.where(upd, vmin, mv)
        marg[...] = jnp.where(upd, varg, marg[...])

    @pl.when(kb == NKB - 1)
    def _():
        mv = mval[...]
        mg = marg[...]
        m = jnp.min(mv, axis=1)
        lane = lax.broadcasted_iota(jnp.int32, (QB, 128), 1)
        flat = mg * 128 + lane
        am = jnp.min(jnp.where(mv == m[:, None], flat, jnp.int32(2**30)),
                     axis=1)
        q = q_ref[...]
        q2 = jnp.sum(q * q, axis=1)
        dist_ref[0, 0, :] = q2 + m
        am_ref[0, 0, :] = am
        qid = qi * QB + lax.broadcasted_iota(jnp.int32, (QB,), 0)
        fidx_ref[0, 0, :] = (qid % HW) * (K // 16) + am // 16


def _distances(qp, keys):
    return pl.pallas_call(
        _dist_kernel,
        grid=(NQB, NKB),
        in_specs=[
            pl.BlockSpec((QB, D), lambda i, j: (i, 0)),
            pl.BlockSpec((KB, D), lambda i, j: (j, 0)),
        ],
        out_specs=[
            pl.BlockSpec((1, 1, QB), lambda i, j: (i, 0, 0)),
            pl.BlockSpec((1, 1, QB), lambda i, j: (i, 0, 0)),
            pl.BlockSpec((1, 1, QB), lambda i, j: (i, 0, 0)),
        ],
        out_shape=[
            jax.ShapeDtypeStruct((NQB, 1, QB), jnp.float32),
            jax.ShapeDtypeStruct((NQB, 1, QB), jnp.int32),
            jax.ShapeDtypeStruct((NQB, 1, QB), jnp.int32),
        ],
        scratch_shapes=[
            pltpu.VMEM((1, K), jnp.float32),
            pltpu.VMEM((QB, 128), jnp.float32),
            pltpu.VMEM((QB, 128), jnp.int32),
        ],
    )(qp, keys)


_NC, _NS = 2, 16
_NW = _NC * _NS
GPAD = 3584
_PER_W = GPAD // _NW


def _sc_gather_kernel(table_hbm, fidx_hbm, out_hbm, idx_v, rows_v, sem):
    wid = lax.axis_index("s") * _NC + lax.axis_index("c")
    base = wid * _PER_W
    pltpu.sync_copy(fidx_hbm.at[pl.ds(base, _PER_W)], idx_v)
    pltpu.async_copy(table_hbm.at[idx_v], rows_v, sem).wait()
    pltpu.sync_copy(rows_v, out_hbm.at[pl.ds(base, _PER_W)])


def _sc_gather(table16, fidx_pad):
    mesh = plsc.VectorSubcoreMesh(core_axis_name="c", subcore_axis_name="s")
    f = functools.partial(
        pl.kernel,
        mesh=mesh,
        out_type=jax.ShapeDtypeStruct((GPAD, 16), jnp.float32),
        scratch_types=[
            pltpu.VMEM((_PER_W,), jnp.int32),
            pltpu.VMEM((_PER_W, 16), jnp.float32),
            pltpu.SemaphoreType.DMA,
        ],
        compiler_params=pltpu.CompilerParams(use_tc_tiling_on_sc=False),
    )(_sc_gather_kernel)
    return f(table16, fidx_pad)


def _post_kernel(dist_ref, rows_ref, am_ref, a_ref, score_ref, ds_ref, ps_ref):
    a = a_ref[...]
    x = dist_ref[0]
    rows = rows_ref[0]
    l16 = am_ref[0] % 16
    lane = lax.broadcasted_iota(jnp.int32, (H, W, 16), 2)
    p = jnp.sum(jnp.where(lane == l16[:, :, None], rows, 0.0), axis=2)
    score_ref[...] = jnp.full((1, 1, 1), jnp.max(x), jnp.float32)
    for src, dst in ((x, ds_ref), (p, ps_ref)):
        t = lax.dot_general(a, src, dimension_numbers=(((1,), (0,)), ((), ())),
                            preferred_element_type=jnp.float32,
                            precision=lax.Precision.HIGHEST)
        o = lax.dot_general(t, a, dimension_numbers=(((1,), (1,)), ((), ())),
                            preferred_element_type=jnp.float32,
                            precision=lax.Precision.HIGHEST)
        mn = jnp.min(o)
        mx = jnp.max(o)
        dst[0] = (o - mn) / (mx - mn)


def _postprocess(dist3, rows3, am3, a_mat):
    return pl.pallas_call(
        _post_kernel,
        grid=(B,),
        in_specs=[
            pl.BlockSpec((1, H, W), lambda i: (i, 0, 0)),
            pl.BlockSpec((1, H, W, 16), lambda i: (i, 0, 0, 0)),
            pl.BlockSpec((1, H, W), lambda i: (i, 0, 0)),
            pl.BlockSpec((TARGET, H), lambda i: (0, 0)),
        ],
        out_specs=[
            pl.BlockSpec((1, 1, 1), lambda i: (i, 0, 0)),
            pl.BlockSpec((1, TARGET, TARGET), lambda i: (i, 0, 0)),
            pl.BlockSpec((1, TARGET, TARGET), lambda i: (i, 0, 0)),
        ],
        out_shape=[
            jax.ShapeDtypeStruct((B, 1, 1), jnp.float32),
            jax.ShapeDtypeStruct((B, TARGET, TARGET), jnp.float32),
            jax.ShapeDtypeStruct((B, TARGET, TARGET), jnp.float32),
        ],
    )(dist3, rows3, am3, a_mat)


def kernel(queries, keys, similar_map):
    dist5, am5, fidx5 = _distances(queries, keys)
    dist = dist5.reshape(Q)
    fidx = fidx5.reshape(Q)
    fidx_pad = jnp.pad(fidx, (0, GPAD - Q))
    table16 = similar_map.transpose(1, 2, 0).reshape(HW * K // 16, 16)
    rows = _sc_gather(table16, fidx_pad)[:Q]
    dist3 = dist.reshape(B, H, W)
    rows3 = rows.reshape(B, H, W, 16)
    am3 = am5.reshape(Q).reshape(B, H, W)
    a_mat = jnp.asarray(_A_NP)
    score3, ds, ps = _postprocess(dist3, rows3, am3, a_mat)
    return (score3.reshape(B), ds, ps)
